# baseline (speedup 1.0000x reference)
import jax
import jax.numpy as jnp
from jax import lax
from jax.experimental import pallas as pl
from jax.experimental.pallas import tpu as pltpu
from jax.experimental.pallas import tpu_sc as plsc

M, D, N = 100000, 128, 500000
NC, NS = 2, 16
NW = NC * NS
B = 160
G = 80
NG = B // G
NCHUNKS = N // B
NCPW = -(-NCHUNKS // NW)
L = 16
NBUF = 5
LA = 3
MAIN = NCPW - LA
assert MAIN % NBUF == 0


def _sc_body(idx_hbm, t_hbm, lu_hbm, table_hbm, w_hbm, b_hbm, out_hbm,
             *scratch):
    idx_v = scratch[0:NBUF]
    t_v = scratch[NBUF:2 * NBUF]
    lu_v = scratch[2 * NBUF:3 * NBUF]
    rows_v = scratch[3 * NBUF:4 * NBUF]
    w_v, b_v = scratch[4 * NBUF], scratch[4 * NBUF + 1]
    fsem = scratch[4 * NBUF + 2:5 * NBUF + 2]
    gsem = scratch[5 * NBUF + 2:6 * NBUF + 2]
    wsem = scratch[6 * NBUF + 2:7 * NBUF + 2]
    cid = lax.axis_index("c")
    sid = lax.axis_index("s")
    wid = sid * NC + cid

    pltpu.sync_copy(w_hbm, w_v)
    pltpu.sync_copy(b_hbm, b_v)
    w_regs = [w_v[pl.ds(L * i, L)] for i in range(D // L)]
    b_regs = [b_v[pl.ds(L * i, L)] + 1.0 for i in range(D // L)]

    def chunk_of(j):
        jc = jnp.minimum(j, NCPW - 1)
        return jnp.minimum(wid + jc * NW, NCHUNKS - 1)

    def issue_fetch(j, s):
        c = chunk_of(j)
        pltpu.async_copy(idx_hbm.at[c], idx_v[s], fsem[s])
        pltpu.async_copy(t_hbm.at[c], t_v[s], fsem[s])

    def wait_fetch(s):
        pltpu.make_async_copy(idx_hbm.at[0], idx_v[s], fsem[s]).wait()
        pltpu.make_async_copy(t_hbm.at[0], t_v[s], fsem[s]).wait()

    def issue_gather(s):
        for g in range(NG):
            pltpu.async_copy(table_hbm.at[idx_v[s].at[pl.ds(g * G, G)]],
                             rows_v[s].at[pl.ds(g * G, G)], gsem[s])
            pltpu.async_copy(lu_hbm.at[idx_v[s].at[pl.ds(g * G, G)]],
                             lu_v[s].at[pl.ds(g * G, G)], gsem[s])

    def wait_gather(s):
        pltpu.make_async_copy(table_hbm.at[pl.ds(0, B)], rows_v[s],
                              gsem[s]).wait()
        pltpu.make_async_copy(lu_hbm.at[pl.ds(0, B)], lu_v[s],
                              gsem[s]).wait()

    def issue_wb(j, s):
        c = chunk_of(j)
        pltpu.async_copy(rows_v[s], out_hbm.at[pl.ds(c * B, B)], wsem[s])

    def wait_wb(s):
        pltpu.make_async_copy(rows_v[s], out_hbm.at[pl.ds(0, B)],
                              wsem[s]).wait()

    def compute(s):
        def group_body(gi, carry):
            r0 = gi * L
            td16 = t_v[s][pl.ds(r0, L)] - lu_v[s][pl.ds(r0, L)]
            for rr in range(L):
                td = td16[rr]
                ri = r0 + rr
                for dc in range(D // L):
                    sl = pl.ds(dc * L, L)
                    rows_v[s][ri, sl] = (
                        rows_v[s][ri, sl] * (td * w_regs[dc] + b_regs[dc]))
            return carry
        lax.fori_loop(0, B // L, group_body, 0)

    def run_iter(i, s, with_wsem):
        sL = (s + LA) % NBUF
        wait_fetch(sL)
        if with_wsem:
            wait_wb(sL)
        issue_gather(sL)
        wait_gather(s)
        compute(s)
        issue_wb(i, s)
        issue_fetch(i + NBUF, s)

    for s in range(NBUF):
        issue_fetch(s, s)
    for s in range(LA):
        wait_fetch(s)
        issue_gather(s)

    for s in range(NBUF):
        run_iter(s, s, s >= NBUF - LA)

    def block_body(k, carry):
        i0 = k * NBUF
        for s in range(NBUF):
            run_iter(i0 + s, s, True)
        return carry
    lax.fori_loop(1, MAIN // NBUF, block_body, 0)

    for i in range(MAIN, NCPW):
        s = i % NBUF
        wait_gather(s)
        compute(s)
        issue_wb(i, s)

    for s in range(LA, NBUF):
        wait_fetch(s)
    for s in range(NBUF):
        wait_wb(s)


@jax.jit
def _sc_embed(idx2, t2, last_update, memory_embeds, w1, b):
    mesh = plsc.VectorSubcoreMesh(core_axis_name="c", subcore_axis_name="s")
    scratch = (
        [pltpu.VMEM((B,), jnp.int32) for _ in range(NBUF)]
        + [pltpu.VMEM((B,), jnp.float32) for _ in range(NBUF)]
        + [pltpu.VMEM((B,), jnp.float32) for _ in range(NBUF)]
        + [pltpu.VMEM((B, D), jnp.float32) for _ in range(NBUF)]
        + [pltpu.VMEM((D,), jnp.float32), pltpu.VMEM((D,), jnp.float32)]
        + [pltpu.SemaphoreType.DMA for _ in range(3 * NBUF)]
    )
    f = pl.kernel(
        _sc_body,
        out_type=jax.ShapeDtypeStruct((N, D), jnp.float32),
        mesh=mesh,
        scratch_types=scratch,
    )
    return f(idx2, t2, last_update, memory_embeds, w1, b)


def kernel(memory_embeds, last_update, idx, t, W, b):
    idx2 = idx.astype(jnp.int32).reshape(NCHUNKS, B)
    t2 = t.reshape(NCHUNKS, B)
    w1 = W.reshape(D)
    return _sc_embed(idx2, t2, last_update, memory_embeds, w1, b)


# speedup vs baseline: 1.0047x; 1.0047x over previous
import jax
import jax.numpy as jnp
from jax import lax
from jax.experimental import pallas as pl
from jax.experimental.pallas import tpu as pltpu
from jax.experimental.pallas import tpu_sc as plsc

M, D, N = 100000, 128, 500000
NC, NS = 2, 16
NW = NC * NS
B = 160
G = 80
NG = B // G
NCHUNKS = N // B
NCPW = -(-NCHUNKS // NW)
L = 16
NBUF = 4
LA = 2
MAIN = NCPW - LA
assert MAIN % NBUF == 0


def _sc_body(idx_hbm, t_hbm, lu_hbm, table_hbm, w_hbm, b_hbm, out_hbm,
             *scratch):
    idx_v = scratch[0:NBUF]
    t_v = scratch[NBUF:2 * NBUF]
    lu_v = scratch[2 * NBUF:3 * NBUF]
    rows_v = scratch[3 * NBUF:4 * NBUF]
    w_v, b_v = scratch[4 * NBUF], scratch[4 * NBUF + 1]
    fsem = scratch[4 * NBUF + 2:5 * NBUF + 2]
    gsem = scratch[5 * NBUF + 2:6 * NBUF + 2]
    wsem = scratch[6 * NBUF + 2:7 * NBUF + 2]
    cid = lax.axis_index("c")
    sid = lax.axis_index("s")
    wid = sid * NC + cid

    pltpu.sync_copy(w_hbm, w_v)
    pltpu.sync_copy(b_hbm, b_v)
    w_regs = [w_v[pl.ds(L * i, L)] for i in range(D // L)]
    b_regs = [b_v[pl.ds(L * i, L)] + 1.0 for i in range(D // L)]

    def chunk_of(j):
        jc = jnp.minimum(j, NCPW - 1)
        return jnp.minimum(wid + jc * NW, NCHUNKS - 1)

    def issue_fetch(j, s):
        c = chunk_of(j)
        pltpu.async_copy(idx_hbm.at[c], idx_v[s], fsem[s])
        pltpu.async_copy(t_hbm.at[c], t_v[s], fsem[s])

    def wait_fetch(s):
        pltpu.make_async_copy(idx_hbm.at[0], idx_v[s], fsem[s]).wait()
        pltpu.make_async_copy(t_hbm.at[0], t_v[s], fsem[s]).wait()

    def issue_gather(s):
        for g in range(NG):
            pltpu.async_copy(table_hbm.at[idx_v[s].at[pl.ds(g * G, G)]],
                             rows_v[s].at[pl.ds(g * G, G)], gsem[s])
            pltpu.async_copy(lu_hbm.at[idx_v[s].at[pl.ds(g * G, G)]],
                             lu_v[s].at[pl.ds(g * G, G)], gsem[s])

    def wait_gather(s):
        pltpu.make_async_copy(table_hbm.at[pl.ds(0, B)], rows_v[s],
                              gsem[s]).wait()
        pltpu.make_async_copy(lu_hbm.at[pl.ds(0, B)], lu_v[s],
                              gsem[s]).wait()

    def issue_wb(j, s):
        c = chunk_of(j)
        pltpu.async_copy(rows_v[s], out_hbm.at[pl.ds(c * B, B)], wsem[s])

    def wait_wb(s):
        pltpu.make_async_copy(rows_v[s], out_hbm.at[pl.ds(0, B)],
                              wsem[s]).wait()

    def compute(s):
        def group_body(gi, carry):
            r0 = gi * L
            td16 = t_v[s][pl.ds(r0, L)] - lu_v[s][pl.ds(r0, L)]
            for rr in range(L):
                td = td16[rr]
                ri = r0 + rr
                for dc in range(D // L):
                    sl = pl.ds(dc * L, L)
                    rows_v[s][ri, sl] = (
                        rows_v[s][ri, sl] * (td * w_regs[dc] + b_regs[dc]))
            return carry
        lax.fori_loop(0, B // L, group_body, 0)

    def run_iter(i, s, with_wsem):
        sL = (s + LA) % NBUF
        wait_fetch(sL)
        if with_wsem:
            wait_wb(sL)
        issue_gather(sL)
        wait_gather(s)
        compute(s)
        issue_wb(i, s)
        issue_fetch(i + NBUF, s)

    for s in range(NBUF):
        issue_fetch(s, s)
    for s in range(LA):
        wait_fetch(s)
        issue_gather(s)

    for s in range(NBUF):
        run_iter(s, s, s >= NBUF - LA)

    def block_body(k, carry):
        i0 = k * NBUF
        for s in range(NBUF):
            run_iter(i0 + s, s, True)
        return carry
    lax.fori_loop(1, MAIN // NBUF, block_body, 0)

    for i in range(MAIN, NCPW):
        s = i % NBUF
        wait_gather(s)
        compute(s)
        issue_wb(i, s)

    for s in range(LA, NBUF):
        wait_fetch(s)
    for s in range(NBUF):
        wait_wb(s)


@jax.jit
def _sc_embed(idx2, t2, last_update, memory_embeds, w1, b):
    mesh = plsc.VectorSubcoreMesh(core_axis_name="c", subcore_axis_name="s")
    scratch = (
        [pltpu.VMEM((B,), jnp.int32) for _ in range(NBUF)]
        + [pltpu.VMEM((B,), jnp.float32) for _ in range(NBUF)]
        + [pltpu.VMEM((B,), jnp.float32) for _ in range(NBUF)]
        + [pltpu.VMEM((B, D), jnp.float32) for _ in range(NBUF)]
        + [pltpu.VMEM((D,), jnp.float32), pltpu.VMEM((D,), jnp.float32)]
        + [pltpu.SemaphoreType.DMA for _ in range(3 * NBUF)]
    )
    f = pl.kernel(
        _sc_body,
        out_type=jax.ShapeDtypeStruct((N, D), jnp.float32),
        mesh=mesh,
        scratch_types=scratch,
    )
    return f(idx2, t2, last_update, memory_embeds, w1, b)


def kernel(memory_embeds, last_update, idx, t, W, b):
    idx2 = idx.astype(jnp.int32).reshape(NCHUNKS, B)
    t2 = t.reshape(NCHUNKS, B)
    w1 = W.reshape(D)
    return _sc_embed(idx2, t2, last_update, memory_embeds, w1, b)
